# baseline (speedup 1.0000x reference)
import functools

import jax
import jax.numpy as jnp
from jax import lax
from jax.experimental import pallas as pl
from jax.experimental.pallas import tpu as pltpu, tpu_sc as plsc

N = 50000
E = 1600000
NC = 2
NS = 16
NTILE = NC * NS
ROWS_PER_TILE = 392
EROWS = ROWS_PER_TILE * NTILE
EPAD = EROWS * 128 - E
NROWS = 50048
NPT = NROWS // NS
D = 7
NG = ROWS_PER_TILE // D


def _leaky(v):
    return jnp.where(v >= 0, v, 0.01 * v)


def _make_sc_pass(wp):
    mesh = plsc.VectorSubcoreMesh(core_axis_name="c", subcore_axis_name="s",
                                  num_cores=NC, num_subcores=NS)

    def body(g, src2, dst2, zer, part0, part1, agg_sh, ibs, ibd, rows, *sems):
        gs = sems[:D]
        ss = sems[D:2 * D]
        is_ = sems[2 * D:2 * D + 2]
        id_ = sems[2 * D + 2:2 * D + 4]
        c = lax.axis_index("c")
        s = lax.axis_index("s")
        w = s * NC + c
        pltpu.sync_copy(zer, agg_sh.at[pl.ds(s * NPT, NPT)])
        plsc.subcore_barrier()

        rb = w * ROWS_PER_TILE

        def rslot(d):
            return rows.at[pl.ds(d * 128, 128)]

        def idx_src(blk):
            return src2.at[pl.ds(rb + blk * D, D)]

        def idx_dst(blk):
            return dst2.at[pl.ds(rb + blk * D, D)]

        pltpu.sync_copy(idx_src(0), ibs.at[0])
        pltpu.sync_copy(idx_dst(0), ibd.at[0])
        pltpu.async_copy(idx_src(1), ibs.at[1], is_[1])
        pltpu.async_copy(idx_dst(1), ibd.at[1], id_[1])
        for d in range(D):
            pltpu.async_copy(g.at[ibs.at[0].at[d]], rslot(d), gs[d])

        def pair(i, carry):
            for p in (0, 1):
                gi = i * 2 + p
                for d in range(D):
                    pltpu.make_async_copy(g.at[ibs.at[p].at[d]],
                                          rslot(d), gs[d]).wait()
                    pltpu.async_copy(rslot(d), agg_sh.at[ibd.at[p].at[d]],
                                     ss[d], add=True)

                def gather_next():
                    pltpu.make_async_copy(idx_src(0), ibs.at[1 - p],
                                          is_[1 - p]).wait()
                    pltpu.make_async_copy(idx_dst(0), ibd.at[1 - p],
                                          id_[1 - p]).wait()
                    for d in range(D):
                        pltpu.make_async_copy(rslot(d),
                                              agg_sh.at[pl.ds(0, 128)],
                                              ss[d]).wait()
                        pltpu.async_copy(g.at[ibs.at[1 - p].at[d]],
                                         rslot(d), gs[d])

                if p == 0:
                    gather_next()

                    @pl.when(i < NG // 2 - 1)
                    def _():
                        pltpu.async_copy(idx_src(gi + 2), ibs.at[0], is_[0])
                        pltpu.async_copy(idx_dst(gi + 2), ibd.at[0], id_[0])
                else:
                    @pl.when(i < NG // 2 - 1)
                    def _():
                        gather_next()
                        pltpu.async_copy(idx_src(gi + 2), ibs.at[1], is_[1])
                        pltpu.async_copy(idx_dst(gi + 2), ibd.at[1], id_[1])
            return carry

        lax.fori_loop(0, NG // 2, pair, 0)
        for d in range(D):
            pltpu.make_async_copy(rslot(d), agg_sh.at[pl.ds(0, 128)],
                                  ss[d]).wait()

        plsc.subcore_barrier()
        sl = pl.ds(s * NPT, NPT)

        @pl.when(c == 0)
        def _():
            pltpu.sync_copy(agg_sh.at[sl], part0.at[sl])

        @pl.when(c == 1)
        def _():
            pltpu.sync_copy(agg_sh.at[sl], part1.at[sl])

    return pl.kernel(
        body,
        out_type=(jax.ShapeDtypeStruct((NROWS, wp), jnp.float32),
                  jax.ShapeDtypeStruct((NROWS, wp), jnp.float32)),
        mesh=mesh,
        scratch_types=[
            pltpu.VMEM_SHARED((NROWS, wp), jnp.float32),
            pltpu.VMEM((2, D, 128), jnp.int32),
            pltpu.VMEM((2, D, 128), jnp.int32),
            pltpu.VMEM((D * 128, wp), jnp.float32),
        ] + [pltpu.SemaphoreType.DMA] * (2 * D + 4),
        compiler_params=pltpu.CompilerParams(use_tc_tiling_on_sc=False),
    )


_BLK = 2000


def _full_spec(*shape):
    return pl.BlockSpec(shape, lambda i: tuple(0 for _ in shape))


def _row_spec(wp):
    return pl.BlockSpec((_BLK, wp), lambda i: (i, 0))


def _combine_body(s_ref, p0_ref, p1_ref, inv_ref, sn_ref):
    sn_ref[:] = 0.5 * s_ref[:] + (0.5 * inv_ref[:]) * (p0_ref[:] + p1_ref[:])


def _combine(s, p0, p1, inv):
    return pl.pallas_call(
        _combine_body,
        grid=(N // _BLK,),
        in_specs=[_row_spec(16), _row_spec(16), _row_spec(16), _row_spec(1)],
        out_specs=[_row_spec(16)],
        out_shape=[jax.ShapeDtypeStruct((NROWS, 16), jnp.float32)],
    )(s, p0, p1, inv)[0]


def _prep_body(x_ref, p0_ref, p1_ref, inv_ref, deg_ref, s0_ref):
    deg = jnp.maximum(p0_ref[:, 0:1] + p1_ref[:, 0:1], 1.0)
    inv = 1.0 / deg
    inv_ref[:] = inv
    deg_ref[:] = deg
    h0 = jnp.concatenate([x_ref[:], jnp.zeros((_BLK, 7), jnp.float32)], axis=1)
    s0_ref[:] = h0 * inv


def _prep(x, p0, p1):
    return pl.pallas_call(
        _prep_body,
        grid=(N // _BLK,),
        in_specs=[_row_spec(9), _row_spec(16), _row_spec(16)],
        out_specs=[_row_spec(1), _row_spec(1), _row_spec(16)],
        out_shape=[jax.ShapeDtypeStruct((N, 1), jnp.float32),
                   jax.ShapeDtypeStruct((N, 1), jnp.float32),
                   jax.ShapeDtypeStruct((NROWS, 16), jnp.float32)],
    )(x, p0, p1)


def _assemble_body(s1_ref, s2_ref, s4_ref, s8_ref, ta_ref, tb_ref):
    b0 = jnp.abs(s1_ref[:] - s2_ref[:])[:, :9]
    b1 = jnp.abs(s2_ref[:] - s4_ref[:])[:, :9]
    b2 = jnp.abs(s4_ref[:] - s8_ref[:])[:, :9]
    t0 = jnp.concatenate([b0, b1, b2, jnp.zeros((_BLK, 5), jnp.float32)],
                         axis=1)
    ta_ref[:] = t0[:, :16]
    tb_ref[:] = t0[:, 16:]


def _assemble(s1, s2, s4, s8):
    return pl.pallas_call(
        _assemble_body,
        grid=(N // _BLK,),
        in_specs=[_row_spec(16)] * 4,
        out_specs=[_row_spec(16), _row_spec(16)],
        out_shape=[jax.ShapeDtypeStruct((NROWS, 16), jnp.float32),
                   jax.ShapeDtypeStruct((NROWS, 16), jnp.float32)],
    )(s1, s2, s4, s8)


def _mlp_body(x_ref, s1_ref, s2_ref, s4_ref, s8_ref, s16_ref,
              ta2_ref, tb2_ref, ta4_ref, tb4_ref,
              ta8_ref, tb8_ref, ta16_ref, tb16_ref, deg_ref,
              W1_ref, b1_ref, W2_ref, b2_ref, W3_ref, b3_ref,
              We_ref, be_ref, Wc_ref, bc_ref, emb_ref, out_ref):
    deg = deg_ref[:]
    s1_1 = (deg * jnp.abs(s1_ref[:] - s2_ref[:]))[:, :9]
    s1_2 = (deg * jnp.abs(s2_ref[:] - s4_ref[:]))[:, :9]
    s1_3 = (deg * jnp.abs(s4_ref[:] - s8_ref[:]))[:, :9]
    s1_4 = (deg * jnp.abs(s8_ref[:] - s16_ref[:]))[:, :9]
    d24 = deg * jnp.concatenate(
        [jnp.abs(ta2_ref[:] - ta4_ref[:]), jnp.abs(tb2_ref[:] - tb4_ref[:])],
        axis=1)
    d48 = deg * jnp.concatenate(
        [jnp.abs(ta4_ref[:] - ta8_ref[:]), jnp.abs(tb4_ref[:] - tb8_ref[:])],
        axis=1)
    d816 = deg * jnp.concatenate(
        [jnp.abs(ta8_ref[:] - ta16_ref[:]), jnp.abs(tb8_ref[:] - tb16_ref[:])],
        axis=1)
    feat = jnp.concatenate([
        x_ref[:], s1_1, s1_2, s1_3, s1_4,
        d24[:, 0:9],
        d48[:, 0:9], d48[:, 9:18],
        d816[:, 0:9], d816[:, 9:18], d816[:, 18:27],
    ], axis=1)
    h = _leaky(feat)
    h = _leaky(jnp.dot(h, W1_ref[:], preferred_element_type=jnp.float32) + b1_ref[:])
    h = _leaky(jnp.dot(h, W2_ref[:], preferred_element_type=jnp.float32) + b2_ref[:])
    h = jnp.dot(h, W3_ref[:], preferred_element_type=jnp.float32) + b3_ref[:]
    e = jnp.dot(h, We_ref[:], preferred_element_type=jnp.float32) + be_ref[:]
    emb_ref[:] = e
    out_ref[:] = jnp.dot(e, Wc_ref[:], preferred_element_type=jnp.float32) + bc_ref[:]


def _mlp(x, s1, s2, s4, s8, s16, t2, t4, t8, t16, deg,
         W1, b1, W2, b2, W3, b3, We, be, Wc, bc):
    weight_specs = [_full_spec(*a.shape)
                    for a in (W1, b1, W2, b2, W3, b3, We, be, Wc, bc)]
    return pl.pallas_call(
        _mlp_body,
        grid=(N // _BLK,),
        in_specs=([_row_spec(9)] + [_row_spec(16)] * 13 + [_row_spec(1)]
                  + weight_specs),
        out_specs=[_row_spec(32), _row_spec(1)],
        out_shape=[jax.ShapeDtypeStruct((N, 32), jnp.float32),
                   jax.ShapeDtypeStruct((N, 1), jnp.float32)],
    )(x, s1, s2, s4, s8, s16,
      t2[0], t2[1], t4[0], t4[1], t8[0], t8[1], t16[0], t16[1], deg,
      W1, b1, W2, b2, W3, b3, We, be, Wc, bc)


def kernel(x, edge_index, batch, W1, b1, W2, b2, W3, b3, We, be, Wc, bc):
    pad = jnp.full((EPAD,), N, jnp.int32)
    src2 = jnp.concatenate([edge_index[0], pad]).reshape(EROWS, 128)
    dst2 = jnp.concatenate([edge_index[1], pad]).reshape(EROWS, 128)
    zer16 = jnp.zeros((NPT, 16), jnp.float32)
    ones16 = jnp.ones((NROWS, 16), jnp.float32)

    sc16 = _make_sc_pass(16)

    d0, d1 = sc16(ones16, src2, src2, zer16)
    inv, deg, s = _prep(x, d0, d1)

    snaps1 = {}
    for k in range(1, 17):
        p0, p1 = sc16(s, src2, dst2, zer16)
        s = _combine(s, p0, p1, inv)
        if k in (1, 2, 4, 8, 16):
            snaps1[k] = s

    ta, tb = _assemble(snaps1[1], snaps1[2], snaps1[4], snaps1[8])

    snaps2 = {}
    for k in range(1, 17):
        p0a, p1a = sc16(ta, src2, dst2, zer16)
        p0b, p1b = sc16(tb, src2, dst2, zer16)
        ta = _combine(ta, p0a, p1a, inv)
        tb = _combine(tb, p0b, p1b, inv)
        if k in (2, 4, 8, 16):
            snaps2[k] = (ta, tb)

    emb, out = _mlp(x, snaps1[1], snaps1[2], snaps1[4], snaps1[8], snaps1[16],
                    snaps2[2], snaps2[4], snaps2[8], snaps2[16], deg,
                    W1, b1, W2, b2, W3, b3, We, be, Wc, bc)
    return (emb, out)
